# Initial kernel scaffold; baseline (speedup 1.0000x reference)
import jax
import jax.numpy as jnp
from jax.experimental import pallas as pl


def kernel(x, W1, g1, b1, W2, g2, b2, W3, g3, b3, W4, g4, b4, W5, g5, b5, Wt, gt, bt, Woff, Wr, gr, br, Wc, bc):
    raise NotImplementedError("write your pallas kernel here")


# speedup vs baseline: 2.6451x; 2.6451x over previous
import functools
import jax
import jax.numpy as jnp
from jax import lax
from jax.experimental import pallas as pl

_K = 20
_NUM_NODE = 64
_EPS = 1e-5
_NEG = -3.0e38


def _leaky(v):
    return jnp.where(v >= 0, v, 0.2 * v)


def _stage_body(x_ref, w_ref, y_ref, st_ref, *, k):
    x = x_ref[0]
    m = x.shape[1]
    w = w_ref[...]
    o = w.shape[0]
    inner = lax.dot_general(x, x, (((0,), (0,)), ((), ())),
                            preferred_element_type=jnp.float32)
    xx = jnp.sum(x * x, axis=0, keepdims=True)
    p = 2.0 * inner - xx - jnp.transpose(xx, (1, 0))
    iota = lax.broadcasted_iota(jnp.int32, (m, m), 1)

    def body(_, carry):
        p_c, r_c, s1_c, s2_c = carry
        v = jnp.max(p_c, axis=1, keepdims=True)
        eq = p_c >= v
        mi = jnp.min(jnp.where(eq, iota, m), axis=1, keepdims=True)
        h = (iota == mi)
        hf = h.astype(jnp.float32)
        xg = lax.dot_general(x, hf, (((1,), (1,)), ((), ())),
                             preferred_element_type=jnp.float32,
                             precision=lax.Precision.HIGHEST)
        feat = jnp.concatenate([xg - x, x], axis=0)
        y = lax.dot_general(w, feat, (((1,), (0,)), ((), ())),
                            preferred_element_type=jnp.float32)
        r_c = jnp.maximum(r_c, y)
        s1_c = s1_c + jnp.sum(y, axis=1, keepdims=True)
        s2_c = s2_c + jnp.sum(y * y, axis=1, keepdims=True)
        p_c = jnp.where(h, _NEG, p_c)
        return p_c, r_c, s1_c, s2_c

    r0 = jnp.full((o, m), _NEG, jnp.float32)
    s10 = jnp.zeros((o, 1), jnp.float32)
    s20 = jnp.zeros((o, 1), jnp.float32)
    _, r, s1, s2 = lax.fori_loop(0, k, body, (p, r0, s10, s20))

    y_ref[0] = r
    st_ref[0, 0:1, :] = jnp.transpose(s1, (1, 0))
    st_ref[0, 1:2, :] = jnp.transpose(s2, (1, 0))


def _stage(x, w, k):
    b, c, m = x.shape
    o = w.shape[0]
    body = functools.partial(_stage_body, k=k)
    return pl.pallas_call(
        body,
        grid=(b,),
        in_specs=[
            pl.BlockSpec((1, c, m), lambda i: (i, 0, 0)),
            pl.BlockSpec((o, 2 * c), lambda i: (0, 0)),
        ],
        out_specs=[
            pl.BlockSpec((1, o, m), lambda i: (i, 0, 0)),
            pl.BlockSpec((1, 2, o), lambda i: (i, 0, 0)),
        ],
        out_shape=[
            jax.ShapeDtypeStruct((b, o, m), jnp.float32),
            jax.ShapeDtypeStruct((b, 2, o), jnp.float32),
        ],
    )(x, w)


def _bn_act_body(y_ref, ss_ref, o_ref, *, act):
    y = y_ref[0]
    s = jnp.transpose(ss_ref[0:1, :], (1, 0))
    t = jnp.transpose(ss_ref[1:2, :], (1, 0))
    v = y * s + t
    if act == 'leaky':
        v = jnp.where(v >= 0, v, 0.2 * v)
    elif act == 'relu':
        v = jnp.maximum(v, 0.0)
    o_ref[0] = v


def _bn_act(y, ss, act):
    b, o, m = y.shape
    body = functools.partial(_bn_act_body, act=act)
    return pl.pallas_call(
        body,
        grid=(b,),
        in_specs=[
            pl.BlockSpec((1, o, m), lambda i: (i, 0, 0)),
            pl.BlockSpec((2, o), lambda i: (0, 0)),
        ],
        out_specs=pl.BlockSpec((1, o, m), lambda i: (i, 0, 0)),
        out_shape=jax.ShapeDtypeStruct((b, o, m), jnp.float32),
    )(y, ss)


def _scale_shift(stats, gamma, beta, count):
    tot = jnp.sum(stats, axis=0)
    mean = tot[0] / count
    var = tot[1] / count - mean * mean
    s = gamma / jnp.sqrt(var + _EPS)
    t = beta - mean * s
    return jnp.stack([s, t])


def _proj_body(x_ref, w_ref, b_ref, o_ref):
    x = x_ref[0]
    y = lax.dot_general(w_ref[...], x, (((1,), (0,)), ((), ())),
                        preferred_element_type=jnp.float32)
    o_ref[0] = y + jnp.transpose(b_ref[...], (1, 0))


def _proj(x, w, bvec):
    b, c, m = x.shape
    o = w.shape[0]
    return pl.pallas_call(
        _proj_body,
        grid=(b,),
        in_specs=[
            pl.BlockSpec((1, c, m), lambda i: (i, 0, 0)),
            pl.BlockSpec((o, c), lambda i: (0, 0)),
            pl.BlockSpec((1, o), lambda i: (0, 0)),
        ],
        out_specs=pl.BlockSpec((1, o, m), lambda i: (i, 0, 0)),
        out_shape=jax.ShapeDtypeStruct((b, o, m), jnp.float32),
    )(x, w, bvec)


def _final_body(x_ref, w_ref, y_ref, st_ref):
    x = x_ref[0]
    y = lax.dot_general(w_ref[...], x, (((1,), (0,)), ((), ())),
                        preferred_element_type=jnp.float32)
    y_ref[0] = y
    ones = jnp.ones((1, x.shape[1]), jnp.float32)
    st_ref[0, 0:1, :] = lax.dot_general(ones, y, (((1,), (1,)), ((), ())),
                                        preferred_element_type=jnp.float32, precision=lax.Precision.HIGHEST)
    st_ref[0, 1:2, :] = lax.dot_general(ones, y * y, (((1,), (1,)), ((), ())),
                                        preferred_element_type=jnp.float32, precision=lax.Precision.HIGHEST)


def _final(xcat, w5):
    b, c, m = xcat.shape
    o = w5.shape[0]
    return pl.pallas_call(
        _final_body,
        grid=(b,),
        in_specs=[
            pl.BlockSpec((1, c, m), lambda i: (i, 0, 0)),
            pl.BlockSpec((o, c), lambda i: (0, 0)),
        ],
        out_specs=[
            pl.BlockSpec((1, o, m), lambda i: (i, 0, 0)),
            pl.BlockSpec((1, 2, o), lambda i: (i, 0, 0)),
        ],
        out_shape=[
            jax.ShapeDtypeStruct((b, o, m), jnp.float32),
            jax.ShapeDtypeStruct((b, 2, o), jnp.float32),
        ],
    )(xcat, w5)


def _final_apply_body(y_ref, ss_ref, o_ref):
    y = y_ref[0]
    m = y.shape[1]
    s = jnp.transpose(ss_ref[0:1, :], (1, 0))
    t = jnp.transpose(ss_ref[1:2, :], (1, 0))
    v = y * s + t
    v = jnp.where(v >= 0, v, 0.2 * v)
    mx = jnp.max(v, axis=1, keepdims=True)
    av = jnp.sum(v, axis=1, keepdims=True) / m
    o_ref[0, 0:1, :] = jnp.transpose(mx, (1, 0))
    o_ref[0, 1:2, :] = jnp.transpose(av, (1, 0))


def _final_apply(y, ss):
    b, o, m = y.shape
    return pl.pallas_call(
        _final_apply_body,
        grid=(b,),
        in_specs=[
            pl.BlockSpec((1, o, m), lambda i: (i, 0, 0)),
            pl.BlockSpec((2, o), lambda i: (0, 0)),
        ],
        out_specs=pl.BlockSpec((1, 2, o), lambda i: (i, 0, 0)),
        out_shape=jax.ShapeDtypeStruct((b, 2, o), jnp.float32),
    )(y, ss)


def _bgather(a, i):
    return jax.vmap(lambda aa, ii: aa[ii])(a, i)


def _fps(loc_t, n):
    npts = loc_t.shape[1]

    def single(p):
        def body(i, carry):
            dists, idxs, last = carry
            d = jnp.sum((p - p[last]) ** 2, axis=-1)
            dists = jnp.minimum(dists, d)
            nxt = jnp.argmax(dists).astype(jnp.int32)
            idxs = idxs.at[i].set(nxt)
            return (dists, idxs, nxt)

        dists0 = jnp.full((npts,), 1e10, dtype=p.dtype)
        idxs0 = jnp.zeros((n,), dtype=jnp.int32)
        _, idxs, _ = lax.fori_loop(1, n, body, (dists0, idxs0, jnp.int32(0)))
        return idxs

    return jax.vmap(single)(loc_t)


def _query_ball(radius, nsample, xyz, new_xyz):
    npts = xyz.shape[1]
    sqr = jnp.sum((new_xyz[:, :, None, :] - xyz[:, None, :, :]) ** 2, axis=-1)
    gidx = jnp.broadcast_to(jnp.arange(npts), sqr.shape)
    gidx = jnp.where(sqr > radius * radius, npts, gidx)
    gidx = jnp.sort(gidx, axis=-1)[:, :, :nsample]
    first = gidx[:, :, :1]
    return jnp.where(gidx == npts, first, gidx)


def _bn_jax(x, gamma, beta, axes):
    mu = jnp.mean(x, axis=axes, keepdims=True)
    v = jnp.var(x, axis=axes, keepdims=True)
    shape = [1] * x.ndim
    shape[1] = -1
    return (x - mu) / jnp.sqrt(v + _EPS) * gamma.reshape(shape) + beta.reshape(shape)


def _adapt_layer(fea, loc, p):
    loc_t = jnp.transpose(loc, (0, 2, 1))
    fea_t = jnp.transpose(fea, (0, 2, 1))
    fidx = _fps(loc_t, _NUM_NODE)
    fploc = _bgather(loc_t, fidx)
    fpfea = _bgather(fea_t, fidx)
    gidx = _query_ball(0.3, _NUM_NODE, loc_t, fploc)
    gfea = _bgather(fea_t, gidx) - fpfea[:, :, None, :]
    gfea_cf = jnp.transpose(gfea, (0, 3, 1, 2))
    seman = jnp.tanh(jnp.einsum('oc,bcsk->bosk', p['Woff'], gfea_cf))
    gloc = _bgather(loc_t, gidx) - fploc[:, :, None, :]
    gloc_cf = jnp.transpose(gloc, (0, 3, 1, 2))
    node_off = jnp.mean(seman * gloc_cf, axis=-1)
    node_loc = jnp.transpose(fploc, (0, 2, 1)) + node_off
    nf = jnp.einsum('oc,bcsk->bosk', p['Wt'], gfea_cf)
    nf = jax.nn.relu(_bn_jax(nf, p['gt'], p['bt'], (0, 2, 3)))
    node_fea = jnp.max(nf, axis=-1)
    nl_t = jnp.transpose(node_loc, (0, 2, 1))
    d = jnp.sum((loc_t[:, :, None, :] - nl_t[:, None, :, :]) ** 2, axis=-1)
    negd, i3 = lax.top_k(-d, 3)
    w = 1.0 / (-negd + 1e-8)
    w = w / jnp.sum(w, axis=-1, keepdims=True)
    gath = _bgather(jnp.transpose(node_fea, (0, 2, 1)), i3)
    interp = jnp.sum(w[..., None] * gath, axis=2)
    interp_cf = jnp.transpose(interp, (0, 2, 1))
    res = jnp.einsum('oc,bcn->bon', p['Wr'], fea)
    res = jax.nn.relu(_bn_jax(res, p['gr'], p['br'], (0, 2)))
    x_ = jnp.concatenate([interp_cf, res], axis=1)
    return x_, node_fea


def kernel(x, W1, g1, b1, W2, g2, b2, W3, g3, b3, W4, g4, b4, W5, g5, b5,
           Wt, gt, bt, Woff, Wr, gr, br, Wc, bc):
    xs = x[..., 0]
    b, _, n = xs.shape
    cnt = jnp.float32(b * n * _K)

    y1, st1 = _stage(xs, W1, _K)
    x1 = _bn_act(y1, _scale_shift(st1, g1, b1, cnt), 'leaky')
    y2, st2 = _stage(x1, W2, _K)
    x2 = _bn_act(y2, _scale_shift(st2, g2, b2, cnt), 'leaky')
    p = dict(Wt=Wt, gt=gt, bt=bt, Woff=Woff, Wr=Wr, gr=gr, br=br)
    x_, node_fea = _adapt_layer(x2, xs, p)
    x2b = _proj(x_, Wc, bc.reshape(1, -1))
    y3, st3 = _stage(x2b, W3, _K)
    x3 = _bn_act(y3, _scale_shift(st3, g3, b3, cnt), 'leaky')
    y4, st4 = _stage(x3, W4, _K)
    x4 = _bn_act(y4, _scale_shift(st4, g4, b4, cnt), 'leaky')
    xcat = jnp.concatenate([x1, x2b, x3, x4], axis=1)
    y5, st5 = _final(xcat, W5)
    pooled = _final_apply(y5, _scale_shift(st5, g5, b5, jnp.float32(b * n)))
    out = jnp.concatenate([pooled[:, 0, :], pooled[:, 1, :]], axis=1)
    return out, node_fea
